# baseline (speedup 1.0000x reference)
import jax
import jax.numpy as jnp
from jax.experimental import pallas as pl
from jax.experimental.pallas import tpu as pltpu

N_USERS = 1_000_000
N_ITEMS = 100_000
DIM = 32

_U_CHUNK = 20000
_I_CHUNK = 10000


def _dma_body(u_in, i_in, u_out, i_out, sem_u, sem_i):
    copies = []
    for c in range(N_USERS // _U_CHUNK):
        copies.append(pltpu.make_async_copy(
            u_in.at[pl.ds(c * _U_CHUNK, _U_CHUNK)],
            u_out.at[pl.ds(c * _U_CHUNK, _U_CHUNK)],
            sem_u,
        ))
    for c in range(N_ITEMS // _I_CHUNK):
        copies.append(pltpu.make_async_copy(
            i_in.at[pl.ds(c * _I_CHUNK, _I_CHUNK)],
            i_out.at[pl.ds(c * _I_CHUNK, _I_CHUNK)],
            sem_i,
        ))
    for c in copies:
        c.start()
    for c in copies:
        c.wait()


def kernel(user_table, item_table):
    return pl.pallas_call(
        _dma_body,
        in_specs=[
            pl.BlockSpec(memory_space=pl.ANY),
            pl.BlockSpec(memory_space=pl.ANY),
        ],
        out_specs=[
            pl.BlockSpec(memory_space=pl.ANY),
            pl.BlockSpec(memory_space=pl.ANY),
        ],
        out_shape=[
            jax.ShapeDtypeStruct((N_USERS, DIM), jnp.float32),
            jax.ShapeDtypeStruct((N_ITEMS, DIM), jnp.float32),
        ],
        scratch_shapes=[pltpu.SemaphoreType.DMA, pltpu.SemaphoreType.DMA],
    )(user_table, item_table)


# speedup vs baseline: 1.0011x; 1.0011x over previous
import jax
import jax.numpy as jnp
from jax.experimental import pallas as pl
from jax.experimental.pallas import tpu as pltpu

N_USERS = 1_000_000
N_ITEMS = 100_000
DIM = 32


def _dma_body(u_in, i_in, u_out, i_out, sem_u, sem_i):
    cu = pltpu.make_async_copy(u_in, u_out, sem_u)
    ci = pltpu.make_async_copy(i_in, i_out, sem_i)
    cu.start()
    ci.start()
    cu.wait()
    ci.wait()


def kernel(user_table, item_table):
    return pl.pallas_call(
        _dma_body,
        in_specs=[
            pl.BlockSpec(memory_space=pl.ANY),
            pl.BlockSpec(memory_space=pl.ANY),
        ],
        out_specs=[
            pl.BlockSpec(memory_space=pl.ANY),
            pl.BlockSpec(memory_space=pl.ANY),
        ],
        out_shape=[
            jax.ShapeDtypeStruct((N_USERS, DIM), jnp.float32),
            jax.ShapeDtypeStruct((N_ITEMS, DIM), jnp.float32),
        ],
        scratch_shapes=[pltpu.SemaphoreType.DMA, pltpu.SemaphoreType.DMA],
    )(user_table, item_table)


# speedup vs baseline: 5.0134x; 5.0079x over previous
import functools

import jax
import jax.numpy as jnp
from jax import lax
from jax.experimental import pallas as pl
from jax.experimental.pallas import tpu as pltpu
from jax.experimental.pallas import tpu_sc as plsc

N_USERS = 1_000_000
N_ITEMS = 100_000
DIM = 32

_LANES = 128

_NC = 2
_NS = 16
_NW = _NC * _NS

_CHUNK = 960
_NB = 4

_U_SLOTS = -(-(-(-N_USERS // _CHUNK)) // _NW)
_I_SLOTS = -(-(-N_ITEMS // _CHUNK) // _NW)

_mesh = plsc.VectorSubcoreMesh(core_axis_name="c", subcore_axis_name="s")


@functools.partial(
    pl.kernel,
    out_type=(
        jax.ShapeDtypeStruct((N_USERS, _LANES), jnp.uint8),
        jax.ShapeDtypeStruct((N_ITEMS, _LANES), jnp.uint8),
    ),
    mesh=_mesh,
    scratch_types=(
        [pltpu.VMEM((_CHUNK, _LANES), jnp.uint8) for _ in range(_NB)]
        + [pltpu.SemaphoreType.DMA for _ in range(2 * _NB)]
    ),
)
def _copy_tables(u_in, i_in, u_out, i_out, *scratch):
    bufs = scratch[:_NB]
    rsems = scratch[_NB:2 * _NB]
    wsems = scratch[2 * _NB:]
    wid = lax.axis_index("s") * _NC + lax.axis_index("c")

    jobs = []
    for t in range(_U_SLOTS):
        base = jnp.minimum((wid + t * _NW) * _CHUNK, N_USERS - _CHUNK)
        jobs.append((u_in, u_out, pl.multiple_of(base, 32)))
    for t in range(_I_SLOTS):
        base = jnp.minimum((wid + t * _NW) * _CHUNK, N_ITEMS - _CHUNK)
        jobs.append((i_in, i_out, pl.multiple_of(base, 32)))
    n = len(jobs)

    def read(j, b):
        src, _, base = jobs[j]
        return pltpu.async_copy(src.at[pl.ds(base, _CHUNK)], bufs[b], rsems[b])

    def write(j, b):
        _, dst, base = jobs[j]
        return pltpu.async_copy(bufs[b], dst.at[pl.ds(base, _CHUNK)], wsems[b])

    reads = [None] * n
    writes = [None] * n
    for b in range(min(_NB, n)):
        reads[b] = read(b, b)
    for j in range(n):
        b = j % _NB
        reads[j].wait()
        writes[j] = write(j, b)
        if j + _NB < n:
            writes[j].wait()
            reads[j + _NB] = read(j + _NB, b)
    for j in range(max(0, n - _NB), n):
        writes[j].wait()


def kernel(user_table, item_table):
    u8_u = lax.bitcast_convert_type(user_table, jnp.uint8).reshape(N_USERS, _LANES)
    u8_i = lax.bitcast_convert_type(item_table, jnp.uint8).reshape(N_ITEMS, _LANES)
    u, i = _copy_tables(u8_u, u8_i)
    u = lax.bitcast_convert_type(u.reshape(N_USERS, DIM, 4), jnp.float32)
    i = lax.bitcast_convert_type(i.reshape(N_ITEMS, DIM, 4), jnp.float32)
    return u, i


# speedup vs baseline: 15.2128x; 3.0344x over previous
import functools

import jax
import jax.numpy as jnp
from jax import lax
from jax.experimental import pallas as pl
from jax.experimental.pallas import tpu as pltpu
from jax.experimental.pallas import tpu_sc as plsc

N_USERS = 1_000_000
N_ITEMS = 100_000
DIM = 32

_NC = 2
_NS = 16
_NW = _NC * _NS

_CHUNK = 240
_NB = 4

_U_SLOTS = -(-(-(-N_USERS // _CHUNK)) // _NW)
_I_SLOTS = -(-(-N_ITEMS // _CHUNK) // _NW)

_mesh = plsc.VectorSubcoreMesh(core_axis_name="c", subcore_axis_name="s")


@functools.partial(
    pl.kernel,
    out_type=(
        jax.ShapeDtypeStruct((N_USERS, DIM), jnp.float32),
        jax.ShapeDtypeStruct((N_ITEMS, DIM), jnp.float32),
    ),
    mesh=_mesh,
    compiler_params=pltpu.CompilerParams(use_tc_tiling_on_sc=False),
    scratch_types=(
        [pltpu.VMEM((_CHUNK, DIM), jnp.float32) for _ in range(_NB)]
        + [pltpu.SemaphoreType.DMA for _ in range(2 * _NB)]
    ),
)
def _copy_tables(u_in, i_in, u_out, i_out, *scratch):
    bufs = scratch[:_NB]
    rsems = scratch[_NB:2 * _NB]
    wsems = scratch[2 * _NB:]
    wid = lax.axis_index("s") * _NC + lax.axis_index("c")

    jobs = []
    for t in range(_U_SLOTS):
        base = jnp.minimum((wid + t * _NW) * _CHUNK, N_USERS - _CHUNK)
        jobs.append((u_in, u_out, pl.multiple_of(base, 8)))
    for t in range(_I_SLOTS):
        base = jnp.minimum((wid + t * _NW) * _CHUNK, N_ITEMS - _CHUNK)
        jobs.append((i_in, i_out, pl.multiple_of(base, 8)))
    n = len(jobs)

    def read(j, b):
        src, _, base = jobs[j]
        return pltpu.async_copy(src.at[pl.ds(base, _CHUNK)], bufs[b], rsems[b])

    def write(j, b):
        _, dst, base = jobs[j]
        return pltpu.async_copy(bufs[b], dst.at[pl.ds(base, _CHUNK)], wsems[b])

    reads = [None] * n
    writes = [None] * n
    for b in range(min(_NB, n)):
        reads[b] = read(b, b)
    for j in range(n):
        b = j % _NB
        reads[j].wait()
        writes[j] = write(j, b)
        if j + _NB < n:
            writes[j].wait()
            reads[j + _NB] = read(j + _NB, b)
    for j in range(max(0, n - _NB), n):
        writes[j].wait()


def kernel(user_table, item_table):
    return _copy_tables(user_table, item_table)


# speedup vs baseline: 17.5011x; 1.1504x over previous
import functools

import jax
import jax.numpy as jnp
from jax import lax
from jax.experimental import pallas as pl
from jax.experimental.pallas import tpu as pltpu
from jax.experimental.pallas import tpu_sc as plsc

N_USERS = 1_000_000
N_ITEMS = 100_000
DIM = 32

_NC = 2
_NS = 16
_NW = _NC * _NS

_CHUNK = 240
_NB = 4

_U_SLOTS = -(-(-(-N_USERS // _CHUNK)) // _NW)

_I_BLOCK = 4000

_mesh = plsc.VectorSubcoreMesh(core_axis_name="c", subcore_axis_name="s")


@functools.partial(
    pl.kernel,
    out_type=jax.ShapeDtypeStruct((N_USERS, DIM), jnp.float32),
    mesh=_mesh,
    scratch_types=(
        [pltpu.VMEM((_CHUNK, DIM), jnp.float32) for _ in range(_NB)]
        + [pltpu.SemaphoreType.DMA for _ in range(2 * _NB)]
    ),
)
def _copy_user(u_in, u_out, *scratch):
    bufs = scratch[:_NB]
    rsems = scratch[_NB:2 * _NB]
    wsems = scratch[2 * _NB:]
    wid = lax.axis_index("s") * _NC + lax.axis_index("c")

    bases = []
    for t in range(_U_SLOTS):
        base = jnp.minimum((wid + t * _NW) * _CHUNK, N_USERS - _CHUNK)
        bases.append(pl.multiple_of(base, 8))
    n = len(bases)

    def read(j, b):
        return pltpu.async_copy(
            u_in.at[pl.ds(bases[j], _CHUNK)], bufs[b], rsems[b])

    def write(j, b):
        return pltpu.async_copy(
            bufs[b], u_out.at[pl.ds(bases[j], _CHUNK)], wsems[b])

    reads = [None] * n
    writes = [None] * n
    for b in range(min(_NB, n)):
        reads[b] = read(b, b)
    for j in range(n):
        b = j % _NB
        reads[j].wait()
        writes[j] = write(j, b)
        if j + _NB < n:
            writes[j].wait()
            reads[j + _NB] = read(j + _NB, b)
    for j in range(max(0, n - _NB), n):
        writes[j].wait()


def _tc_copy_body(src_ref, dst_ref):
    dst_ref[...] = src_ref[...]


def kernel(user_table, item_table):
    item_out = pl.pallas_call(
        _tc_copy_body,
        grid=(N_ITEMS // _I_BLOCK,),
        in_specs=[pl.BlockSpec((_I_BLOCK, DIM), lambda i: (i, 0))],
        out_specs=pl.BlockSpec((_I_BLOCK, DIM), lambda i: (i, 0)),
        out_shape=jax.ShapeDtypeStruct((N_ITEMS, DIM), jnp.float32),
    )(item_table)
    user_out = _copy_user(user_table)
    return user_out, item_out
